# baseline (speedup 1.0000x reference)
import jax
import jax.numpy as jnp
from jax.experimental import pallas as pl
from jax.experimental.pallas import tpu as pltpu

_B, _S, _D = 1, 2048, 1024
_H = 16
_HD = _D // _H
_CHUNK = 256
_NC = _S // _CHUNK
_TOPK = 5
_SCALE = _HD ** -0.5
_HP = _H // 2


def _qkv_kernel(x_ref, wq_ref, wk_ref, wv_ref, q_ref, k_ref, v_ref):
    x = x_ref[...]
    dn = (((1,), (1,)), ((), ()))
    q_ref[...] = jax.lax.dot_general(x, wq_ref[...], dn,
                                     preferred_element_type=jnp.float32) * _SCALE
    k_ref[...] = jax.lax.dot_general(x, wk_ref[...], dn,
                                     preferred_element_type=jnp.float32)
    v_ref[...] = jax.lax.dot_general(x, wv_ref[...], dn,
                                     preferred_element_type=jnp.float32)


def _select(sims):
    col = jax.lax.broadcasted_iota(jnp.int32, (_CHUNK, _NC), 1)
    cols = []
    for c in range(_NC):
        sc = sims[:, c:c + 1]
        beats = (sims > sc) | ((sims == sc) & (col < c))
        rank = jnp.sum(beats.astype(jnp.int32), axis=1, keepdims=True)
        cols.append((rank < _TOPK).astype(jnp.float32))
    return jnp.concatenate(cols, axis=1)


def _attn_kernel(q_ref, k_ref, vx_ref, ck_ref, o_ref, acc_ref):
    qc = pl.program_id(1)
    dn_t = (((1,), (1,)), ((), ()))
    dn_n = (((1,), (0,)), ((), ()))

    ri = jax.lax.broadcasted_iota(jnp.int32, (_CHUNK, _CHUNK), 0)
    ci = jax.lax.broadcasted_iota(jnp.int32, (_CHUNK, _CHUNK), 1)
    tri = (ci <= ri).astype(jnp.float32)

    outs = []
    for h2 in range(2):
        qh = q_ref[:, h2 * _HD:(h2 + 1) * _HD]
        kh = k_ref[:, h2 * _HD:(h2 + 1) * _HD]
        vxh = vx_ref[:, h2 * 2 * _HD:(h2 + 1) * 2 * _HD]
        ckh = ck_ref[0][:, h2 * _HD:(h2 + 1) * _HD]

        sims = jax.lax.dot_general(qh, ckh, dn_t,
                                   preferred_element_type=jnp.float32)
        sel = _select(sims)

        acc_ref[...] = jnp.zeros((_CHUNK, 2 * _HD), jnp.float32)
        for kc in range(_NC):
            @pl.when(kc <= qc)
            def _():
                s = jax.lax.dot_general(
                    qh, kh[kc * _CHUNK:(kc + 1) * _CHUNK], dn_t,
                    preferred_element_type=jnp.float32)
                mask = jnp.where(kc == qc, tri, 1.0) * sel[:, kc:kc + 1]
                p = jnp.exp(s) * mask
                acc_ref[...] += jax.lax.dot_general(
                    p, vxh[kc * _CHUNK:(kc + 1) * _CHUNK], dn_n,
                    preferred_element_type=jnp.float32)

        acc = acc_ref[...]
        pv = acc[:, :_HD]
        l = acc[:, _HD:_HD + 1]

        ones8 = jnp.ones((8, _S), jnp.float32)
        sv = jax.lax.dot_general(ones8, vxh, dn_n,
                                 preferred_element_type=jnp.float32)
        vmean = sv[0:1, :_HD] * (1.0 / _S)
        deg = (l == 0.0).astype(jnp.float32)
        safe_l = l + deg
        outs.append(pv / safe_l * (1.0 - deg) + vmean * deg)

    o_ref[...] = jnp.concatenate(outs, axis=1)


def _oproj_kernel(a_ref, wo_ref, o_ref):
    o_ref[...] = jax.lax.dot_general(
        a_ref[...], wo_ref[...], (((1,), (1,)), ((), ())),
        preferred_element_type=jnp.float32)


def kernel(x, Wq, Wk, Wv, Wo):
    x2 = x.reshape(_S, _D)
    f32 = jnp.float32

    q, k, v = pl.pallas_call(
        _qkv_kernel,
        grid=(_NC,),
        in_specs=[
            pl.BlockSpec((_CHUNK, _D), lambda i: (i, 0)),
            pl.BlockSpec((_D, _D), lambda i: (0, 0)),
            pl.BlockSpec((_D, _D), lambda i: (0, 0)),
            pl.BlockSpec((_D, _D), lambda i: (0, 0)),
        ],
        out_specs=[
            pl.BlockSpec((_CHUNK, _D), lambda i: (i, 0)),
            pl.BlockSpec((_CHUNK, _D), lambda i: (i, 0)),
            pl.BlockSpec((_CHUNK, _D), lambda i: (i, 0)),
        ],
        out_shape=[jax.ShapeDtypeStruct((_S, _D), f32)] * 3,
    )(x2, Wq, Wk, Wv)

    K4 = k.reshape(_B, _S, _H, _HD).transpose(0, 2, 1, 3)
    ck = K4.reshape(_B, _H, _NC, _CHUNK, _HD).mean(axis=3)[0]
    ckp = ck.reshape(_HP, 2, _NC, _HD).transpose(0, 2, 1, 3).reshape(
        _HP, _NC, 2 * _HD)

    v4 = v.reshape(_S, _H, _HD)
    vx = jnp.concatenate(
        [v4, jnp.ones((_S, _H, _HD), f32)], axis=2).reshape(_S, 2 * _D)

    attn = pl.pallas_call(
        _attn_kernel,
        grid=(_HP, _NC),
        in_specs=[
            pl.BlockSpec((_CHUNK, 2 * _HD), lambda hp, qc: (qc, hp)),
            pl.BlockSpec((_S, 2 * _HD), lambda hp, qc: (0, hp)),
            pl.BlockSpec((_S, 4 * _HD), lambda hp, qc: (0, hp)),
            pl.BlockSpec((1, _NC, 2 * _HD), lambda hp, qc: (hp, 0, 0)),
        ],
        out_specs=pl.BlockSpec((_CHUNK, 2 * _HD), lambda hp, qc: (qc, hp)),
        out_shape=jax.ShapeDtypeStruct((_S, _D), f32),
        scratch_shapes=[pltpu.VMEM((_CHUNK, 2 * _HD), f32)],
    )(q, k, vx, ckp)

    out = pl.pallas_call(
        _oproj_kernel,
        grid=(_NC,),
        in_specs=[
            pl.BlockSpec((_CHUNK, _D), lambda i: (i, 0)),
            pl.BlockSpec((_D, _D), lambda i: (0, 0)),
        ],
        out_specs=pl.BlockSpec((_CHUNK, _D), lambda i: (i, 0)),
        out_shape=jax.ShapeDtypeStruct((_S, _D), f32),
    )(attn, Wo)

    return out.reshape(_B, _S, _D)


# speedup vs baseline: 2.3462x; 2.3462x over previous
import jax
import jax.numpy as jnp
from jax.experimental import pallas as pl

_B, _S, _D = 1, 2048, 1024
_H = 16
_HD = _D // _H
_CHUNK = 256
_NC = _S // _CHUNK
_TOPK = 5
_SCALE = _HD ** -0.5
_HP = _H // 2


def _qkv_kernel(x_ref, wq_ref, wk_ref, wv_ref, q_ref, k_ref, v_ref):
    x = x_ref[...]
    dn = (((1,), (1,)), ((), ()))
    q_ref[...] = jax.lax.dot_general(x, wq_ref[...], dn,
                                     preferred_element_type=jnp.float32) * _SCALE
    k_ref[...] = jax.lax.dot_general(x, wk_ref[...], dn,
                                     preferred_element_type=jnp.float32)
    v_ref[...] = jax.lax.dot_general(x, wv_ref[...], dn,
                                     preferred_element_type=jnp.float32)


def _routing_kernel(q_ref, ckbd_ref, sel_ref):
    sims = jax.lax.dot_general(q_ref[...], ckbd_ref[...],
                               (((1,), (0,)), ((), ())),
                               preferred_element_type=jnp.float32)

    lane = jax.lax.broadcasted_iota(jnp.int32, (_S, _H * _NC), 1)
    c_lane = lane % _NC
    rank = jnp.zeros((_S, _H * _NC), jnp.int32)
    for r in range(1, _NC):
        left = jnp.roll(sims, -r, axis=1)
        right = jnp.roll(sims, _NC - r, axis=1)
        same = (c_lane + r) < _NC
        w = jnp.where(same, left, right)
        beats = (w > sims) | ((w == sims) & (~same))
        rank = rank + beats.astype(jnp.int32)
    sel = (rank < _TOPK).astype(jnp.float32)

    for hp in range(_HP):
        band = sel[:, hp * 2 * _NC:(hp + 1) * 2 * _NC]
        sel_ref[hp] = jnp.concatenate(
            [band, jnp.zeros((_S, _H * _NC - 2 * _NC), jnp.float32)], axis=1)


def _attn_kernel(q_ref, k_ref, vx_ref, sel_ref, o_ref):
    qc = pl.program_id(1)
    dn_t = (((1,), (1,)), ((), ()))
    dn_n = (((1,), (0,)), ((), ()))

    ri = jax.lax.broadcasted_iota(jnp.int32, (_CHUNK, _CHUNK), 0)
    ci = jax.lax.broadcasted_iota(jnp.int32, (_CHUNK, _CHUNK), 1)
    tri = (ci <= ri).astype(jnp.float32)
    sel = sel_ref[0]

    outs = []
    for h2 in range(2):
        qh = q_ref[:, h2 * _HD:(h2 + 1) * _HD]
        kh = k_ref[:, h2 * _HD:(h2 + 1) * _HD]
        vxh = vx_ref[:, h2 * 2 * _HD:(h2 + 1) * 2 * _HD]

        s = jax.lax.dot_general(qh, kh, dn_t,
                                preferred_element_type=jnp.float32)
        e = jnp.exp(s)

        tiles = []
        for kc in range(_NC):
            lane = h2 * _NC + kc
            allowed = sel[:, lane:lane + 1]
            gate = jnp.where(kc == qc, tri,
                             jnp.float32(1.0) * (kc < qc))
            tiles.append(e[:, kc * _CHUNK:(kc + 1) * _CHUNK]
                         * (allowed * gate))
        p = jnp.concatenate(tiles, axis=1)

        acc = jax.lax.dot_general(p, vxh, dn_n,
                                  preferred_element_type=jnp.float32)
        pv = acc[:, :_HD]
        l = acc[:, _HD:_HD + 1]

        ones8 = jnp.ones((8, _S), jnp.float32)
        sv = jax.lax.dot_general(ones8, vxh, dn_n,
                                 preferred_element_type=jnp.float32)
        vmean = sv[0:1, :_HD] * (1.0 / _S)
        deg = (l == 0.0).astype(jnp.float32)
        safe_l = l + deg
        outs.append(pv / safe_l * (1.0 - deg) + vmean * deg)

    o_ref[...] = jnp.concatenate(outs, axis=1)


def _oproj_kernel(a_ref, wo_ref, o_ref):
    o_ref[...] = jax.lax.dot_general(
        a_ref[...], wo_ref[...], (((1,), (1,)), ((), ())),
        preferred_element_type=jnp.float32)


def kernel(x, Wq, Wk, Wv, Wo):
    x2 = x.reshape(_S, _D)
    f32 = jnp.float32

    q, k, v = pl.pallas_call(
        _qkv_kernel,
        grid=(_NC,),
        in_specs=[
            pl.BlockSpec((_CHUNK, _D), lambda i: (i, 0)),
            pl.BlockSpec((_D, _D), lambda i: (0, 0)),
            pl.BlockSpec((_D, _D), lambda i: (0, 0)),
            pl.BlockSpec((_D, _D), lambda i: (0, 0)),
        ],
        out_specs=[
            pl.BlockSpec((_CHUNK, _D), lambda i: (i, 0)),
            pl.BlockSpec((_CHUNK, _D), lambda i: (i, 0)),
            pl.BlockSpec((_CHUNK, _D), lambda i: (i, 0)),
        ],
        out_shape=[jax.ShapeDtypeStruct((_S, _D), f32)] * 3,
    )(x2, Wq, Wk, Wv)

    K4 = k.reshape(_B, _S, _H, _HD).transpose(0, 2, 1, 3)
    ck = K4.reshape(_B, _H, _NC, _CHUNK, _HD).mean(axis=3)[0]
    eye = jnp.eye(_H, dtype=f32)
    ckbd = (ck.transpose(0, 2, 1)[:, :, None, :]
            * eye[:, None, :, None]).reshape(_D, _H * _NC)

    selp = pl.pallas_call(
        _routing_kernel,
        grid=(1,),
        in_specs=[
            pl.BlockSpec((_S, _D), lambda i: (0, 0)),
            pl.BlockSpec((_D, _H * _NC), lambda i: (0, 0)),
        ],
        out_specs=pl.BlockSpec((_HP, _S, _H * _NC), lambda i: (0, 0, 0)),
        out_shape=jax.ShapeDtypeStruct((_HP, _S, _H * _NC), f32),
    )(q, ckbd)

    v4 = v.reshape(_S, _H, _HD)
    vx = jnp.concatenate(
        [v4, jnp.ones((_S, _H, _HD), f32)], axis=2).reshape(_S, 2 * _D)

    attn = pl.pallas_call(
        _attn_kernel,
        grid=(_HP, _NC),
        in_specs=[
            pl.BlockSpec((_CHUNK, 2 * _HD), lambda hp, qc: (qc, hp)),
            pl.BlockSpec((_S, 2 * _HD), lambda hp, qc: (0, hp)),
            pl.BlockSpec((_S, 4 * _HD), lambda hp, qc: (0, hp)),
            pl.BlockSpec((1, _CHUNK, _H * _NC), lambda hp, qc: (hp, qc, 0)),
        ],
        out_specs=pl.BlockSpec((_CHUNK, 2 * _HD), lambda hp, qc: (qc, hp)),
        out_shape=jax.ShapeDtypeStruct((_S, _D), f32),
    )(q, k, vx, selp)

    out = pl.pallas_call(
        _oproj_kernel,
        grid=(_NC,),
        in_specs=[
            pl.BlockSpec((_CHUNK, _D), lambda i: (i, 0)),
            pl.BlockSpec((_D, _D), lambda i: (0, 0)),
        ],
        out_specs=pl.BlockSpec((_CHUNK, _D), lambda i: (i, 0)),
        out_shape=jax.ShapeDtypeStruct((_S, _D), f32),
    )(attn, Wo)

    return out.reshape(_B, _S, _D)


# speedup vs baseline: 2.4051x; 1.0251x over previous
import jax
import jax.numpy as jnp
from jax.experimental import pallas as pl

_B, _S, _D = 1, 2048, 1024
_H = 16
_HD = _D // _H
_CHUNK = 256
_NC = _S // _CHUNK
_TOPK = 5
_SCALE = _HD ** -0.5
_HP = _H // 2
_SLO = _S - _CHUNK


def _qkv_kernel(x_ref, wq_ref, wk_ref, wv_ref, q_ref, k_ref, vx_ref):
    x = x_ref[...]
    dn = (((1,), (1,)), ((), ()))
    q_ref[...] = jax.lax.dot_general(x, wq_ref[...], dn,
                                     preferred_element_type=jnp.float32) * _SCALE
    k_ref[...] = jax.lax.dot_general(x, wk_ref[...], dn,
                                     preferred_element_type=jnp.float32)
    v = jax.lax.dot_general(x, wv_ref[...], dn,
                            preferred_element_type=jnp.float32)
    ones = jnp.ones((_CHUNK, _HD), jnp.float32)
    pieces = []
    for h in range(_H):
        pieces.append(v[:, h * _HD:(h + 1) * _HD])
        pieces.append(ones)
    vx_ref[...] = jnp.concatenate(pieces, axis=1)


def _routing_kernel(q_ref, ckbd_ref, sel_ref):
    sims = jax.lax.dot_general(q_ref[...], ckbd_ref[...],
                               (((1,), (0,)), ((), ())),
                               preferred_element_type=jnp.float32)

    lane = jax.lax.broadcasted_iota(jnp.int32, (_S, _H * _NC), 1)
    c_lane = lane % _NC
    rank = jnp.zeros((_S, _H * _NC), jnp.int32)
    for r in range(1, _NC):
        left = jnp.roll(sims, -r, axis=1)
        right = jnp.roll(sims, _NC - r, axis=1)
        same = (c_lane + r) < _NC
        w = jnp.where(same, left, right)
        beats = (w > sims) | ((w == sims) & (~same))
        rank = rank + beats.astype(jnp.int32)
    sel = (rank < _TOPK).astype(jnp.float32)

    for hp in range(_HP):
        band = sel[:, hp * 2 * _NC:(hp + 1) * 2 * _NC]
        sel_ref[hp] = jnp.concatenate(
            [band, jnp.zeros((_S, _H * _NC - 2 * _NC), jnp.float32)], axis=1)


def _attn_kernel(q_ref, k_ref, vx_ref, sel_ref, o_ref):
    qc = pl.program_id(1)
    dn_t = (((1,), (1,)), ((), ()))
    dn_n = (((1,), (0,)), ((), ()))

    ri = jax.lax.broadcasted_iota(jnp.int32, (_CHUNK, _CHUNK), 0)
    ci = jax.lax.broadcasted_iota(jnp.int32, (_CHUNK, _CHUNK), 1)
    tri = (ci <= ri).astype(jnp.float32)
    sel = sel_ref[0]
    c_lane = jax.lax.broadcasted_iota(
        jnp.int32, (_CHUNK, _H * _NC), 1) % _NC
    sel_lo = sel * (c_lane < qc)
    col8 = jax.lax.broadcasted_iota(jnp.int32, (_CHUNK, _NC), 1)
    diag8 = (col8 == qc).astype(jnp.float32)

    outs = []
    for h2 in range(2):
        qh = q_ref[:, h2 * _HD:(h2 + 1) * _HD]
        kh = k_ref[:_SLO, h2 * _HD:(h2 + 1) * _HD]
        vxh = vx_ref[:, h2 * 2 * _HD:(h2 + 1) * 2 * _HD]

        s = jax.lax.dot_general(qh, kh, dn_t,
                                preferred_element_type=jnp.float32)
        e = jnp.exp(s)

        kd = k_ref[pl.ds(qc * _CHUNK, _CHUNK), h2 * _HD:(h2 + 1) * _HD]
        vd = vx_ref[pl.ds(qc * _CHUNK, _CHUNK),
                    h2 * 2 * _HD:(h2 + 1) * 2 * _HD]
        sd = jax.lax.dot_general(qh, kd, dn_t,
                                 preferred_element_type=jnp.float32)
        sel_h2 = sel[:, h2 * _NC:(h2 + 1) * _NC]
        a_d = jnp.sum(sel_h2 * diag8, axis=1, keepdims=True)
        pd = jnp.exp(sd) * (tri * a_d)
        acc = jax.lax.dot_general(pd, vd, dn_n,
                                  preferred_element_type=jnp.float32)

        for kc in range(_NC - 1):
            lane = h2 * _NC + kc
            t = e[:, kc * _CHUNK:(kc + 1) * _CHUNK] \
                * sel_lo[:, lane:lane + 1]
            acc = acc + jax.lax.dot_general(
                t, vxh[kc * _CHUNK:(kc + 1) * _CHUNK], dn_n,
                preferred_element_type=jnp.float32)

        pv = acc[:, :_HD]
        l = acc[:, _HD:_HD + 1]

        ones8 = jnp.ones((8, _S), jnp.float32)
        sv = jax.lax.dot_general(ones8, vxh, dn_n,
                                 preferred_element_type=jnp.float32)
        vmean = sv[0:1, :_HD] * (1.0 / _S)
        deg = (l == 0.0).astype(jnp.float32)
        safe_l = l + deg
        outs.append(pv / safe_l * (1.0 - deg) + vmean * deg)

    o_ref[...] = jnp.concatenate(outs, axis=1)


def _oproj_kernel(a_ref, wo_ref, o_ref):
    o_ref[...] = jax.lax.dot_general(
        a_ref[...], wo_ref[...], (((1,), (1,)), ((), ())),
        preferred_element_type=jnp.float32)


def kernel(x, Wq, Wk, Wv, Wo):
    x2 = x.reshape(_S, _D)
    f32 = jnp.float32

    q, k, vx = pl.pallas_call(
        _qkv_kernel,
        grid=(_NC,),
        in_specs=[
            pl.BlockSpec((_CHUNK, _D), lambda i: (i, 0)),
            pl.BlockSpec((_D, _D), lambda i: (0, 0)),
            pl.BlockSpec((_D, _D), lambda i: (0, 0)),
            pl.BlockSpec((_D, _D), lambda i: (0, 0)),
        ],
        out_specs=[
            pl.BlockSpec((_CHUNK, _D), lambda i: (i, 0)),
            pl.BlockSpec((_CHUNK, _D), lambda i: (i, 0)),
            pl.BlockSpec((_CHUNK, 2 * _D), lambda i: (i, 0)),
        ],
        out_shape=[
            jax.ShapeDtypeStruct((_S, _D), f32),
            jax.ShapeDtypeStruct((_S, _D), f32),
            jax.ShapeDtypeStruct((_S, 2 * _D), f32),
        ],
    )(x2, Wq, Wk, Wv)

    K4 = k.reshape(_B, _S, _H, _HD).transpose(0, 2, 1, 3)
    ck = K4.reshape(_B, _H, _NC, _CHUNK, _HD).mean(axis=3)[0]
    eye = jnp.eye(_H, dtype=f32)
    ckbd = (ck.transpose(0, 2, 1)[:, :, None, :]
            * eye[:, None, :, None]).reshape(_D, _H * _NC)

    selp = pl.pallas_call(
        _routing_kernel,
        grid=(1,),
        in_specs=[
            pl.BlockSpec((_S, _D), lambda i: (0, 0)),
            pl.BlockSpec((_D, _H * _NC), lambda i: (0, 0)),
        ],
        out_specs=pl.BlockSpec((_HP, _S, _H * _NC), lambda i: (0, 0, 0)),
        out_shape=jax.ShapeDtypeStruct((_HP, _S, _H * _NC), f32),
    )(q, ckbd)

    attn = pl.pallas_call(
        _attn_kernel,
        grid=(_HP, _NC),
        in_specs=[
            pl.BlockSpec((_CHUNK, 2 * _HD), lambda hp, qc: (qc, hp)),
            pl.BlockSpec((_S, 2 * _HD), lambda hp, qc: (0, hp)),
            pl.BlockSpec((_S, 4 * _HD), lambda hp, qc: (0, hp)),
            pl.BlockSpec((1, _CHUNK, _H * _NC), lambda hp, qc: (hp, qc, 0)),
        ],
        out_specs=pl.BlockSpec((_CHUNK, 2 * _HD), lambda hp, qc: (qc, hp)),
        out_shape=jax.ShapeDtypeStruct((_S, _D), f32),
    )(q, k, vx, selp)

    out = pl.pallas_call(
        _oproj_kernel,
        grid=(_NC,),
        in_specs=[
            pl.BlockSpec((_CHUNK, _D), lambda i: (i, 0)),
            pl.BlockSpec((_D, _D), lambda i: (0, 0)),
        ],
        out_specs=pl.BlockSpec((_CHUNK, _D), lambda i: (i, 0)),
        out_shape=jax.ShapeDtypeStruct((_S, _D), f32),
    )(attn, Wo)

    return out.reshape(_B, _S, _D)


# speedup vs baseline: 2.6343x; 1.0953x over previous
import jax
import jax.numpy as jnp
from jax.experimental import pallas as pl

_B, _S, _D = 1, 2048, 1024
_H = 16
_HD = _D // _H
_CHUNK = 256
_NC = _S // _CHUNK
_TOPK = 5
_SCALE = _HD ** -0.5
_HP = _H // 2
_SLO = _S - _CHUNK


def _qkv_kernel(x_ref, wq_ref, wk_ref, wv_ref, q_ref, k_ref, vx_ref):
    x = x_ref[...]
    dn = (((1,), (1,)), ((), ()))
    q = jax.lax.dot_general(x, wq_ref[...], dn,
                            preferred_element_type=jnp.float32) * _SCALE
    q_ref[...] = q.astype(jnp.bfloat16)
    k_ref[...] = jax.lax.dot_general(x, wk_ref[...], dn,
                                     preferred_element_type=jnp.float32)
    v = jax.lax.dot_general(x, wv_ref[...], dn,
                            preferred_element_type=jnp.float32)
    vb = v.astype(jnp.bfloat16)
    ones = jnp.ones((_CHUNK, _HD), jnp.bfloat16)
    pieces = []
    for h in range(_H):
        pieces.append(vb[:, h * _HD:(h + 1) * _HD])
        pieces.append(ones)
    vx_ref[...] = jnp.concatenate(pieces, axis=1)


def _routing_kernel(q_ref, ckbd_ref, sel_ref):
    sims = jax.lax.dot_general(q_ref[...],
                               ckbd_ref[...].astype(jnp.bfloat16),
                               (((1,), (0,)), ((), ())),
                               preferred_element_type=jnp.float32)

    lane = jax.lax.broadcasted_iota(jnp.int32, (_S, _H * _NC), 1)
    c_lane = lane % _NC
    rank = jnp.zeros((_S, _H * _NC), jnp.int32)
    for r in range(1, _NC):
        left = jnp.roll(sims, -r, axis=1)
        right = jnp.roll(sims, _NC - r, axis=1)
        same = (c_lane + r) < _NC
        w = jnp.where(same, left, right)
        beats = (w > sims) | ((w == sims) & (~same))
        rank = rank + beats.astype(jnp.int32)
    sel = (rank < _TOPK).astype(jnp.float32)

    for hp in range(_HP):
        band = sel[:, hp * 2 * _NC:(hp + 1) * 2 * _NC]
        sel_ref[hp] = jnp.concatenate(
            [band, jnp.zeros((_S, _H * _NC - 2 * _NC), jnp.float32)], axis=1)


def _attn_kernel(q_ref, k_ref, vx_ref, sel_ref, o_ref):
    qc = pl.program_id(1)
    dn_t = (((1,), (1,)), ((), ()))
    dn_n = (((1,), (0,)), ((), ()))

    ri = jax.lax.broadcasted_iota(jnp.int32, (_CHUNK, _CHUNK), 0)
    ci = jax.lax.broadcasted_iota(jnp.int32, (_CHUNK, _CHUNK), 1)
    tri = (ci <= ri).astype(jnp.float32)
    sel = sel_ref[0]
    c_lane = jax.lax.broadcasted_iota(
        jnp.int32, (_CHUNK, _H * _NC), 1) % _NC
    sel_lo = sel * (c_lane < qc)
    col8 = jax.lax.broadcasted_iota(jnp.int32, (_CHUNK, _NC), 1)
    diag8 = (col8 == qc).astype(jnp.float32)

    outs = []
    for h2 in range(2):
        qh = q_ref[:, h2 * _HD:(h2 + 1) * _HD]
        kh = k_ref[:_SLO, h2 * _HD:(h2 + 1) * _HD].astype(jnp.bfloat16)
        vxh = vx_ref[:, h2 * 2 * _HD:(h2 + 1) * 2 * _HD]

        s = jax.lax.dot_general(qh, kh, dn_t,
                                preferred_element_type=jnp.float32)
        e = jnp.exp(s)

        tiles = []
        for kc in range(_NC - 1):
            lane = h2 * _NC + kc
            tiles.append(e[:, kc * _CHUNK:(kc + 1) * _CHUNK]
                         * sel_lo[:, lane:lane + 1])
        p = jnp.concatenate(tiles, axis=1).astype(jnp.bfloat16)

        acc = jax.lax.dot_general(p, vxh[:_SLO], dn_n,
                                  preferred_element_type=jnp.float32)

        kd = k_ref[pl.ds(qc * _CHUNK, _CHUNK),
                   h2 * _HD:(h2 + 1) * _HD].astype(jnp.bfloat16)
        vd = vx_ref[pl.ds(qc * _CHUNK, _CHUNK),
                    h2 * 2 * _HD:(h2 + 1) * 2 * _HD]
        sd = jax.lax.dot_general(qh, kd, dn_t,
                                 preferred_element_type=jnp.float32)
        sel_h2 = sel[:, h2 * _NC:(h2 + 1) * _NC]
        a_d = jnp.sum(sel_h2 * diag8, axis=1, keepdims=True)
        pd = (jnp.exp(sd) * (tri * a_d)).astype(jnp.bfloat16)
        acc = acc + jax.lax.dot_general(pd, vd, dn_n,
                                        preferred_element_type=jnp.float32)

        pv = acc[:, :_HD]
        l = acc[:, _HD:_HD + 1]

        ones8 = jnp.ones((8, _S), jnp.bfloat16)
        sv = jax.lax.dot_general(ones8, vx_ref[:, h2 * 2 * _HD:
                                               (h2 + 1) * 2 * _HD],
                                 dn_n, preferred_element_type=jnp.float32)
        vmean = sv[0:1, :_HD] * (1.0 / _S)
        deg = (l == 0.0).astype(jnp.float32)
        safe_l = l + deg
        outs.append(pv / safe_l * (1.0 - deg) + vmean * deg)

    o_ref[...] = jnp.concatenate(outs, axis=1)


def _oproj_kernel(a_ref, wo_ref, o_ref):
    o_ref[...] = jax.lax.dot_general(
        a_ref[...], wo_ref[...], (((1,), (1,)), ((), ())),
        preferred_element_type=jnp.float32)


def kernel(x, Wq, Wk, Wv, Wo):
    x2 = x.reshape(_S, _D)
    f32 = jnp.float32
    bf16 = jnp.bfloat16

    q, k, vx = pl.pallas_call(
        _qkv_kernel,
        grid=(_NC,),
        in_specs=[
            pl.BlockSpec((_CHUNK, _D), lambda i: (i, 0)),
            pl.BlockSpec((_D, _D), lambda i: (0, 0)),
            pl.BlockSpec((_D, _D), lambda i: (0, 0)),
            pl.BlockSpec((_D, _D), lambda i: (0, 0)),
        ],
        out_specs=[
            pl.BlockSpec((_CHUNK, _D), lambda i: (i, 0)),
            pl.BlockSpec((_CHUNK, _D), lambda i: (i, 0)),
            pl.BlockSpec((_CHUNK, 2 * _D), lambda i: (i, 0)),
        ],
        out_shape=[
            jax.ShapeDtypeStruct((_S, _D), bf16),
            jax.ShapeDtypeStruct((_S, _D), f32),
            jax.ShapeDtypeStruct((_S, 2 * _D), bf16),
        ],
    )(x2, Wq, Wk, Wv)

    K4 = k.reshape(_B, _S, _H, _HD).transpose(0, 2, 1, 3)
    ck = K4.reshape(_B, _H, _NC, _CHUNK, _HD).mean(axis=3)[0]
    eye = jnp.eye(_H, dtype=f32)
    ckbd = (ck.transpose(0, 2, 1)[:, :, None, :]
            * eye[:, None, :, None]).reshape(_D, _H * _NC)

    selp = pl.pallas_call(
        _routing_kernel,
        grid=(1,),
        in_specs=[
            pl.BlockSpec((_S, _D), lambda i: (0, 0)),
            pl.BlockSpec((_D, _H * _NC), lambda i: (0, 0)),
        ],
        out_specs=pl.BlockSpec((_HP, _S, _H * _NC), lambda i: (0, 0, 0)),
        out_shape=jax.ShapeDtypeStruct((_HP, _S, _H * _NC), f32),
    )(q, ckbd)

    attn = pl.pallas_call(
        _attn_kernel,
        grid=(_HP, _NC),
        in_specs=[
            pl.BlockSpec((_CHUNK, 2 * _HD), lambda hp, qc: (qc, hp)),
            pl.BlockSpec((_S, 2 * _HD), lambda hp, qc: (0, hp)),
            pl.BlockSpec((_S, 4 * _HD), lambda hp, qc: (0, hp)),
            pl.BlockSpec((1, _CHUNK, _H * _NC), lambda hp, qc: (hp, qc, 0)),
        ],
        out_specs=pl.BlockSpec((_CHUNK, 2 * _HD), lambda hp, qc: (qc, hp)),
        out_shape=jax.ShapeDtypeStruct((_S, _D), f32),
    )(q, k, vx, selp)

    out = pl.pallas_call(
        _oproj_kernel,
        grid=(_NC,),
        in_specs=[
            pl.BlockSpec((_CHUNK, _D), lambda i: (i, 0)),
            pl.BlockSpec((_D, _D), lambda i: (0, 0)),
        ],
        out_specs=pl.BlockSpec((_CHUNK, _D), lambda i: (i, 0)),
        out_shape=jax.ShapeDtypeStruct((_S, _D), f32),
    )(attn, Wo)

    return out.reshape(_B, _S, _D)
